# baseline (speedup 1.0000x reference)
import functools

import jax
import jax.numpy as jnp
import numpy as np
from jax import lax
from jax.experimental import pallas as pl
from jax.experimental.pallas import tpu as pltpu
from jax.experimental.pallas import tpu_sc as plsc

_FIELD_DIMS = [100000] * 26
_OFFSETS = np.array((0,) + tuple(np.cumsum(_FIELD_DIMS)[:-1]), dtype=np.int32)
_TOTAL = int(sum(_FIELD_DIMS))
_B = 16384
_F = 26
_D = 16

_NC = 2
_NS = 16
_NW = _NC * _NS
_ROWS_PER_W = _B // _NW
_C = 128
_NCHUNK = _ROWS_PER_W // _C


def _tree_sum(vs):
    while len(vs) > 1:
        vs = [vs[i] + vs[i + 1] for i in range(0, len(vs) - 1, 2)] + (
            [vs[-1]] if len(vs) % 2 else [])
    return vs[0]


def _fm_kernel(table_hbm, idx_hbm, w_hbm, out_hbm,
               idx_v, rows_v, wv_v, out_v, sem_rows, sem_w):
    wid = lax.axis_index("s") * _NC + lax.axis_index("c")
    base = wid * _ROWS_PER_W

    lane = lax.iota(jnp.int32, 16)
    wmask2 = lane >= 6
    lane0 = lane == 0

    for c in range(_NCHUNK):
        cbase = (base + c * _C) * _F
        pltpu.sync_copy(idx_hbm.at[pl.ds(cbase, _C * _F)], idx_v)
        cp_rows = pltpu.async_copy(table_hbm.at[idx_v], rows_v, sem_rows)
        cp_w = pltpu.async_copy(w_hbm.at[idx_v], wv_v, sem_w)
        cp_rows.wait()
        cp_w.wait()

        def body(b, _):
            off = b * _F
            vs = [rows_v[off + f] for f in range(_F)]
            s = _tree_sum(vs)
            ss = _tree_sum([v * v for v in vs])
            u = 0.5 * (s * s - ss)
            wv1 = wv_v[pl.ds(off, 16)]
            wv2 = jnp.where(wmask2, wv_v[pl.ds(off + 10, 16)], 0.0)
            r = lax.reduce_sum(u + wv1 + wv2, (0,))
            plsc.store_scatter(out_v, [jnp.broadcast_to(b, (16,))],
                               jnp.broadcast_to(r, (16,)), mask=lane0)
            return 0

        lax.fori_loop(0, _C, body, 0)
        pltpu.sync_copy(out_v, out_hbm.at[pl.ds(base + c * _C, _C)])


@jax.jit
def _fm(table, idx_flat, w_flat):
    mesh = plsc.VectorSubcoreMesh(core_axis_name="c", subcore_axis_name="s")
    run = functools.partial(
        pl.kernel,
        out_type=jax.ShapeDtypeStruct((_B,), jnp.float32),
        mesh=mesh,
        scratch_types=[
            pltpu.VMEM((_C * _F,), jnp.int32),
            pltpu.VMEM((_C * _F, _D), jnp.float32),
            pltpu.VMEM((_C * _F,), jnp.float32),
            pltpu.VMEM((_C,), jnp.float32),
            pltpu.SemaphoreType.DMA,
            pltpu.SemaphoreType.DMA,
        ],
        compiler_params=pltpu.CompilerParams(
            needs_layout_passes=False, use_tc_tiling_on_sc=False),
    )(_fm_kernel)
    return run(table, idx_flat, w_flat)


def kernel(x, table, w, bias):
    idx = (x + jnp.asarray(_OFFSETS)[None, :]).reshape(-1)
    out = _fm(table, idx, w.reshape(-1))
    return out.reshape(_B, 1) + bias
